# Initial kernel scaffold; baseline (speedup 1.0000x reference)
import jax
import jax.numpy as jnp
from jax.experimental import pallas as pl


def kernel(features, edge_index_u2t, edge_index_t2u, embed_user, W0_u2t, b0_u2t, W0_t2u, b0_t2u, W1_u2t, b1_u2t, W1_t2u, b1_t2u, Wc, bc):
    raise NotImplementedError("write your pallas kernel here")


# speedup vs baseline: 1.1193x; 1.1193x over previous
import jax
import jax.numpy as jnp
from jax import lax
from jax.experimental import pallas as pl
from jax.experimental.pallas import tpu as pltpu
from jax.experimental.pallas import tpu_sc as plsc

N_NODES = 50000
D = 128
E = 300000
N_CLS = 8

NTILES = 16
NCORES = 2
NCHUNK = 8
DC = D // NCHUNK
BATCH = 128
NBATCH = 80
EPT = NBATCH * BATCH
E_PAD = EPT * NTILES * NCORES
EROWS = E_PAD // BATCH
NT = 50176
TB = NT + 8
DUMP = NT
STRIPE = NT // NTILES
BLK = 392


def _seg_sums(tabs, src2d, dst2d):
    def body(*refs):
        tabs_in = refs[:NCHUNK]
        (src_hbm, dst_hbm, sums_o, cnt_o,
         src_st, dst_st, rows, ones, zbuf, acc, sem) = refs[NCHUNK:]
        cid = lax.axis_index("c")
        sid = lax.axis_index("s")
        er0 = (cid * NTILES + sid) * NBATCH

        pltpu.sync_copy(src_hbm.at[pl.ds(er0, NBATCH)], src_st)
        pltpu.sync_copy(dst_hbm.at[pl.ds(er0, NBATCH)], dst_st)

        zf = jnp.zeros((16,), jnp.float32)
        of = jnp.ones((16,), jnp.float32)

        def init_row(r, _):
            for l in range(DC // 16):
                zbuf[r, pl.ds(l * 16, 16)] = zf
                ones[r, pl.ds(l * 16, 16)] = of
            return 0

        lax.fori_loop(0, BATCH, init_row, 0)

        def zero_acc():
            z0 = sid * STRIPE
            for j in range((STRIPE + BATCH - 1) // BATCH + 1):
                s = jnp.minimum(z0 + j * BATCH, TB - BATCH)
                pltpu.sync_copy(zbuf, acc.at[pl.ds(s, BATCH)])
            plsc.subcore_barrier()

        def writeout(dst_view):
            plsc.subcore_barrier()
            w0 = sid * STRIPE
            pltpu.sync_copy(acc.at[pl.ds(w0, STRIPE)],
                            dst_view.at[pl.ds(w0, STRIPE)])
            plsc.subcore_barrier()

        for c in range(NCHUNK):
            tab = tabs_in[c]
            zero_acc()

            def biter(b, _):
                pltpu.async_copy(tab.at[src_st.at[b]], rows, sem).wait()
                pltpu.sync_copy(rows, acc.at[dst_st.at[b]], add=True)
                return 0

            lax.fori_loop(0, NBATCH, biter, 0)
            writeout(sums_o.at[c, cid])

        zero_acc()

        def citer(b, _):
            pltpu.sync_copy(ones, acc.at[dst_st.at[b]], add=True)
            return 0

        lax.fori_loop(0, NBATCH, citer, 0)
        writeout(cnt_o.at[cid])

    k = pl.kernel(
        body,
        out_type=(
            jax.ShapeDtypeStruct((NCHUNK, NCORES, NT, DC), jnp.float32),
            jax.ShapeDtypeStruct((NCORES, NT, DC), jnp.float32),
        ),
        mesh=plsc.VectorSubcoreMesh(core_axis_name="c", subcore_axis_name="s"),
        scratch_types=(
            pltpu.VMEM((NBATCH, BATCH), jnp.int32),
            pltpu.VMEM((NBATCH, BATCH), jnp.int32),
            pltpu.VMEM((BATCH, DC), jnp.float32),
            pltpu.VMEM((BATCH, DC), jnp.float32),
            pltpu.VMEM((BATCH, DC), jnp.float32),
            pltpu.VMEM_SHARED((TB, DC), jnp.float32),
            pltpu.SemaphoreType.DMA,
        ),
        compiler_params=pltpu.CompilerParams(use_tc_tiling_on_sc=False),
    )
    return k(*tabs, src2d, dst2d)


def _stage_in(x, w, b):
    n = x.shape[0]
    blk = 400 if n % 400 == 0 else BLK

    def kern(x_ref, w_ref, b_ref, *outs):
        h = (jnp.dot(x_ref[...], w_ref[...],
                     preferred_element_type=jnp.float32) + b_ref[...])
        for c in range(NCHUNK):
            outs[c][...] = h[:, c * DC:(c + 1) * DC]

    return pl.pallas_call(
        kern,
        grid=(n // blk,),
        in_specs=[
            pl.BlockSpec((blk, D), lambda i: (i, 0)),
            pl.BlockSpec((D, D), lambda i: (0, 0)),
            pl.BlockSpec((1, D), lambda i: (0, 0)),
        ],
        out_specs=[pl.BlockSpec((blk, DC), lambda i: (i, 0))
                   for _ in range(NCHUNK)],
        out_shape=[jax.ShapeDtypeStruct((n, DC), jnp.float32)
                   for _ in range(NCHUNK)],
    )(x, w, b.reshape(1, D))


def _stage_merge(sums, cnt, w, b, relu, split_out):
    d_out = w.shape[1]

    def kern(s_ref, c_ref, w_ref, b_ref, *outs):
        hs = [s_ref[c, 0] + s_ref[c, 1] for c in range(NCHUNK)]
        h = jnp.concatenate(hs, axis=1)
        cnt_b = c_ref[0] + c_ref[1]
        h = h / jnp.maximum(cnt_b[:, 0:1], 1.0)
        if relu:
            h = jnp.where(h >= 0, h, 0.01 * h)
        o = (jnp.dot(h, w_ref[...], preferred_element_type=jnp.float32)
             + b_ref[...])
        if split_out:
            for c in range(NCHUNK):
                outs[c][...] = o[:, c * DC:(c + 1) * DC]
        else:
            outs[0][...] = o

    if split_out:
        out_specs = [pl.BlockSpec((BLK, DC), lambda i: (i, 0))
                     for _ in range(NCHUNK)]
        out_shape = [jax.ShapeDtypeStruct((NT, DC), jnp.float32)
                     for _ in range(NCHUNK)]
    else:
        out_specs = [pl.BlockSpec((BLK, d_out), lambda i: (i, 0))]
        out_shape = [jax.ShapeDtypeStruct((NT, d_out), jnp.float32)]

    return pl.pallas_call(
        kern,
        grid=(NT // BLK,),
        in_specs=[
            pl.BlockSpec((NCHUNK, NCORES, BLK, DC), lambda i: (0, 0, i, 0)),
            pl.BlockSpec((NCORES, BLK, DC), lambda i: (0, i, 0)),
            pl.BlockSpec((D, d_out), lambda i: (0, 0)),
            pl.BlockSpec((1, d_out), lambda i: (0, 0)),
        ],
        out_specs=out_specs,
        out_shape=out_shape,
    )(sums, cnt, w, b.reshape(1, d_out))


def _edges_2d(ei):
    npad = E_PAD - E
    src = jnp.concatenate([ei[0].astype(jnp.int32),
                           jnp.zeros((npad,), jnp.int32)])
    dst = jnp.concatenate([ei[1].astype(jnp.int32),
                           jnp.full((npad,), DUMP, jnp.int32)])
    return src.reshape(EROWS, BATCH), dst.reshape(EROWS, BATCH)


def kernel(features, edge_index_u2t, edge_index_t2u, embed_user,
           W0_u2t, b0_u2t, W0_t2u, b0_t2u,
           W1_u2t, b1_u2t, W1_t2u, b1_t2u, Wc, bc):
    src_t2u, dst_t2u = _edges_2d(edge_index_t2u)
    src_u2t, dst_u2t = _edges_2d(edge_index_u2t)

    wh0 = _stage_in(features, W0_t2u, b0_t2u)
    sums_u, cnt_u = _seg_sums(wh0, src_t2u, dst_t2u)
    wh1 = _stage_merge(sums_u, cnt_u, W1_u2t, b1_u2t, relu=True,
                       split_out=True)
    sums_t, cnt_t = _seg_sums(wh1, src_u2t, dst_u2t)
    out = _stage_merge(sums_t, cnt_t, Wc, bc, relu=False, split_out=False)[0]
    return out[:N_NODES]
